# baseline (speedup 1.0000x reference)
import functools
import numpy as np
import jax
import jax.numpy as jnp
from jax import lax
from jax.experimental import pallas as pl
from jax.experimental.pallas import tpu as pltpu
from jax.experimental.pallas import tpu_sc as plsc

INPUT_H, INPUT_W = 96, 320
UPRATIO = 4
MAX_DEPTH = 80.0
BN_EPS = 1e-5

ROWS_PER_BLK = 48
NBLK = INPUT_H // ROWS_PER_BLK
PIX_BLK = ROWS_PER_BLK * INPUT_W
G_ROWS = ROWS_PER_BLK * 4
UP_H = INPUT_H * UPRATIO
UP_W = INPUT_W * UPRATIO
DS_H = UP_H // 2
DS_W = UP_W // 2

_PREC = jax.lax.Precision.DEFAULT

_NC, _NS, _LANES = 2, 16, 16
_NW = _NC * _NS
_UNITS_PER_W = 4 * INPUT_H // _NW


def _make_tc_body(relu_flags):
    n = len(relu_flags)

    def body(*refs):
        x_ref = refs[0]
        w_refs = refs[1:1 + 2 * n]
        g_ref = refs[1 + 2 * n]

        h = x_ref[0]
        for i in range(n):
            w = w_refs[2 * i][...]
            b = w_refs[2 * i + 1][...]
            h = jnp.dot(w, h, precision=_PREC) + b
            if relu_flags[i]:
                h = jnp.maximum(h, 0.0)
        t0 = jnp.tanh(h[0:1])
        t1 = jnp.tanh(h[1:2])
        c2 = jax.nn.sigmoid(h[2:3])
        d3 = jax.nn.sigmoid(h[3:4]) * MAX_DEPTH
        norm = jnp.sqrt(t0 * t0 + t1 * t1 + c2 * c2)
        inv = 1.0 / jnp.maximum(norm, 1e-12)
        P = jnp.concatenate([t0 * inv, t1 * inv, c2 * inv, d3], axis=0)
        G = jnp.concatenate(
            [P[:, r * INPUT_W:(r + 1) * INPUT_W] for r in range(ROWS_PER_BLK)],
            axis=0)
        g_ref[0] = G

    return body


def _sc_expand_call(plane, table, B):
    f32 = jnp.float32

    @functools.partial(
        pl.kernel,
        out_type=[
            jax.ShapeDtypeStruct((B, 1, UP_H, UP_W), f32),
            jax.ShapeDtypeStruct((B, 1, DS_H, DS_W), f32),
        ],
        scratch_types=[
            pltpu.VMEM((4, INPUT_W), f32),
            pltpu.VMEM((4, 4, _LANES), f32),
            pltpu.VMEM((4, UP_W), f32),
            pltpu.VMEM((2, DS_W), f32),
        ],
        mesh=plsc.VectorSubcoreMesh(core_axis_name="c", subcore_axis_name="s"),
    )
    def sc_expand(plane_hbm, t_hbm, depth_hbm, ds_hbm,
                  prow_v, t_v, dbuf_v, dsbuf_v):
        wid = lax.axis_index("s") * _NC + lax.axis_index("c")

        iota = lax.iota(jnp.int32, _LANES)
        rowdiv = lax.shift_right_logical(iota, 2)
        evidx = (2 * iota) & 15
        odidx = (2 * iota + 1) & 15
        lowm = iota < 8

        def unit_body(k, carry):
            unit = wid * _UNITS_PER_W + k
            b = unit // INPUT_H
            r = unit % INPUT_H
            pltpu.sync_copy(plane_hbm.at[b, pl.ds(4 * r, 4), :], prow_v)
            pltpu.sync_copy(t_hbm.at[b], t_v)

            for j in range(UP_W // _LANES):
                st = min(4 * j, INPUT_W - 16)
                idxj = rowdiv + (4 * j - st)
                a16 = prow_v[0, pl.ds(st, 16)].at[idxj].get(
                    mode="promise_in_bounds")
                b16 = prow_v[1, pl.ds(st, 16)].at[idxj].get(
                    mode="promise_in_bounds")
                c16 = prow_v[2, pl.ds(st, 16)].at[idxj].get(
                    mode="promise_in_bounds")
                d16 = prow_v[3, pl.ds(st, 16)].at[idxj].get(
                    mode="promise_in_bounds")
                for p in range(4):
                    den = a16 * t_v[p, 0, :] + b16 * t_v[p, 1, :] + c16
                    dbuf_v[p, pl.ds(16 * j, 16)] = d16 * t_v[p, 2, :] / den

            ratio16 = t_v[0, 3, :]
            for p2 in range(2):
                oy = 2 * r + p2
                wy = oy.astype(jnp.float32) * (1.0 / (DS_H - 1.0))
                for jc in range(DS_W // _LANES):
                    wx = (iota + 16 * jc).astype(jnp.float32) * (1.0 / (DS_W - 1.0))
                    rowmix = []
                    for row in (2 * p2, 2 * p2 + 1):
                        w0 = dbuf_v[row, pl.ds(32 * jc, 16)]
                        w1 = dbuf_v[row, pl.ds(32 * jc + 16, 16)]
                        ev = jnp.where(
                            lowm,
                            w0.at[evidx].get(mode="promise_in_bounds"),
                            w1.at[evidx].get(mode="promise_in_bounds"))
                        od = jnp.where(
                            lowm,
                            w0.at[odidx].get(mode="promise_in_bounds"),
                            w1.at[odidx].get(mode="promise_in_bounds"))
                        rowmix.append(ev * (1.0 - wx) + od * wx)
                    mix = rowmix[0] * (1.0 - wy) + rowmix[1] * wy
                    dsbuf_v[p2, pl.ds(16 * jc, 16)] = mix * ratio16

            pltpu.sync_copy(dbuf_v, depth_hbm.at[b, 0, pl.ds(4 * r, 4), :])
            pltpu.sync_copy(dsbuf_v, ds_hbm.at[b, 0, pl.ds(2 * r, 2), :])
            return carry

        lax.fori_loop(0, _UNITS_PER_W, unit_body, 0)

    return sc_expand(plane, table)


def kernel(x, focal, downratio, params):
    B, C, H, W = x.shape
    f32 = jnp.float32

    Ws, bs, relu_flags = [], [], []
    for p in params:
        Wm, bv = p['W'], p['b']
        if 'gamma' in p:
            s = p['gamma'] / jnp.sqrt(1.0 + BN_EPS)
            Wm = Wm * s[:, None]
            bv = bv * s + p['beta']
            relu_flags.append(True)
        else:
            relu_flags.append(False)
        Ws.append(Wm.astype(f32))
        bs.append(bv.astype(f32)[:, None])

    xr = x.reshape(B, C, H * W)

    wb_inputs, wb_specs = [], []
    for Wm, bv in zip(Ws, bs):
        wb_inputs += [Wm, bv]
        wb_specs += [
            pl.BlockSpec(Wm.shape, lambda bi, ii: (0, 0)),
            pl.BlockSpec(bv.shape, lambda bi, ii: (0, 0)),
        ]

    plane = pl.pallas_call(
        _make_tc_body(tuple(relu_flags)),
        grid=(B, NBLK),
        in_specs=[
            pl.BlockSpec((1, C, PIX_BLK), lambda bi, ii: (bi, 0, ii)),
            *wb_specs,
        ],
        out_specs=pl.BlockSpec((1, G_ROWS, INPUT_W), lambda bi, ii: (bi, ii, 0)),
        out_shape=jax.ShapeDtypeStruct((B, 4 * INPUT_H, INPUT_W), f32),
    )(xr, *wb_inputs)

    off = (UPRATIO - 1) / 2.0
    u16 = ((np.arange(_LANES, dtype=np.float32) % UPRATIO) - off)[None, None, :] \
        / focal[:, None, None]
    vp = ((np.arange(UPRATIO, dtype=np.float32)) - off)[None, :, None] \
        / focal[:, None, None]
    U = jnp.broadcast_to(u16, (B, 4, _LANES)).astype(f32)
    V = jnp.broadcast_to(vp, (B, 4, _LANES)).astype(f32)
    S = jnp.sqrt(U * U + V * V + 1.0) * (1.0 / MAX_DEPTH)
    ratio = (jnp.asarray(downratio, f32) / 2.0) * jnp.ones((B, 4, _LANES), f32)
    table = jnp.stack([U, V, S, ratio], axis=2)

    depth, ds = _sc_expand_call(plane, table, B)
    return (depth, ds)


# speedup vs baseline: 1.7332x; 1.7332x over previous
import numpy as np
import jax
import jax.numpy as jnp
from jax.experimental import pallas as pl

INPUT_H, INPUT_W = 96, 320
UPRATIO = 4
MAX_DEPTH = 80.0
BN_EPS = 1e-5

ROWS_PER_BLK = 48
NBLK = INPUT_H // ROWS_PER_BLK
PIX_BLK = ROWS_PER_BLK * INPUT_W
D_ROWS = ROWS_PER_BLK * UPRATIO
UP_H = INPUT_H * UPRATIO
UP_W = INPUT_W * UPRATIO
DS_ROWS = D_ROWS // 2
DS_H = UP_H // 2
DS_W = UP_W // 2

_HI = jax.lax.Precision.DEFAULT


def _np_consts():
    S = np.zeros((INPUT_W, UP_W), np.float32)
    S[np.arange(UP_W) // UPRATIO, np.arange(UP_W)] = 1.0
    U = np.zeros((4 * D_ROWS, D_ROWS), np.float32)
    for coeff in range(4):
        for i in range(D_ROWS):
            U[coeff * D_ROWS + i, UPRATIO * (i // UPRATIO) + coeff] = 1.0
    WY = np.zeros((NBLK, DS_ROWS, D_ROWS), np.float32)
    for i in range(NBLK):
        for o in range(DS_ROWS):
            wy = (DS_ROWS * i + o) / (DS_H - 1.0)
            WY[i, o, 2 * o] = 1.0 - wy
            WY[i, o, 2 * o + 1] = wy
    WX = np.zeros((UP_W, DS_W), np.float32)
    for ox in range(DS_W):
        wx = ox / (DS_W - 1.0)
        WX[2 * ox, ox] = 1.0 - wx
        WX[2 * ox + 1, ox] = wx
    return S, U, WY, WX


_S_NP, _U_NP, _WY_NP, _WX_NP = _np_consts()


def _make_body(relu_flags):
    n = len(relu_flags)

    def body(*refs):
        x_ref, u_ref, v_ref = refs[0:3]
        w_refs = refs[3:3 + 2 * n]
        s_ref, u32_ref, wy_ref, wx_ref, d_ref, ds_ref = refs[3 + 2 * n:]

        h = x_ref[0]
        for i in range(n):
            w = w_refs[2 * i][...]
            b = w_refs[2 * i + 1][...]
            h = jnp.dot(w, h, precision=_HI) + b
            if relu_flags[i]:
                h = jnp.maximum(h, 0.0)
        t0 = jnp.tanh(h[0:1])
        t1 = jnp.tanh(h[1:2])
        c2 = jax.nn.sigmoid(h[2:3])
        d3 = jax.nn.sigmoid(h[3:4]) * MAX_DEPTH
        norm = jnp.sqrt(t0 * t0 + t1 * t1 + c2 * c2)
        inv = 1.0 / jnp.maximum(norm, 1e-12)
        P = jnp.concatenate([t0 * inv, t1 * inv, c2 * inv, d3], axis=0)
        G = jnp.concatenate(
            [P[:, r * INPUT_W:(r + 1) * INPUT_W] for r in range(ROWS_PER_BLK)],
            axis=0)
        S = s_ref[...]
        GS = jnp.dot(G, S, precision=_HI)
        R = jnp.dot(u32_ref[...], GS, precision=_HI)
        A, Bc, Cc, Dc = (R[0:D_ROWS], R[D_ROWS:2 * D_ROWS],
                         R[2 * D_ROWS:3 * D_ROWS], R[3 * D_ROWS:4 * D_ROWS])

        u = u_ref[0]
        v = v_ref[0]
        den = A * u + Bc * v + Cc
        num = Dc * jnp.sqrt(u * u + v * v + 1.0)
        dsc = (num / den) * (1.0 / MAX_DEPTH)
        d_ref[0, 0] = dsc
        rm = jnp.dot(wy_ref[0], dsc, precision=_HI)
        ds_ref[0, 0] = jnp.dot(rm, wx_ref[...], precision=_HI)

    return body


def kernel(x, focal, downratio, params):
    B, C, H, W = x.shape
    f32 = jnp.float32

    Ws, bs, relu_flags = [], [], []
    for p in params:
        Wm, bv = p['W'], p['b']
        if 'gamma' in p:
            s = p['gamma'] / jnp.sqrt(1.0 + BN_EPS)
            Wm = Wm * s[:, None]
            bv = bv * s + p['beta']
            relu_flags.append(True)
        else:
            relu_flags.append(False)
        Ws.append(Wm.astype(f32))
        bs.append(bv.astype(f32)[:, None])

    xr = x.reshape(B, C, H * W)
    u_base = (np.arange(UP_W, dtype=np.float32) % UPRATIO) - (UPRATIO - 1) / 2.0
    v_base = (np.arange(D_ROWS, dtype=np.float32) % UPRATIO) - (UPRATIO - 1) / 2.0
    u_all = (u_base[None, None, :] / focal[:, None, None]).astype(f32)
    v_all = (v_base[None, :, None] / focal[:, None, None]).astype(f32)

    ratio = jnp.asarray(downratio, f32) / 2.0
    WXr = jnp.asarray(_WX_NP) * ratio

    wb_inputs, wb_specs = [], []
    for Wm, bv in zip(Ws, bs):
        wb_inputs += [Wm, bv]
        wb_specs += [
            pl.BlockSpec(Wm.shape, lambda bi, ii: (0, 0)),
            pl.BlockSpec(bv.shape, lambda bi, ii: (0, 0)),
        ]

    grid = (B, NBLK)
    out = pl.pallas_call(
        _make_body(tuple(relu_flags)),
        grid=grid,
        in_specs=[
            pl.BlockSpec((1, C, PIX_BLK), lambda bi, ii: (bi, 0, ii)),
            pl.BlockSpec((1, 1, UP_W), lambda bi, ii: (bi, 0, 0)),
            pl.BlockSpec((1, D_ROWS, 1), lambda bi, ii: (bi, 0, 0)),
            *wb_specs,
            pl.BlockSpec((INPUT_W, UP_W), lambda bi, ii: (0, 0)),
            pl.BlockSpec((4 * D_ROWS, D_ROWS), lambda bi, ii: (0, 0)),
            pl.BlockSpec((1, DS_ROWS, D_ROWS), lambda bi, ii: (ii, 0, 0)),
            pl.BlockSpec((UP_W, DS_W), lambda bi, ii: (0, 0)),
        ],
        out_specs=[
            pl.BlockSpec((1, 1, D_ROWS, UP_W), lambda bi, ii: (bi, 0, ii, 0)),
            pl.BlockSpec((1, 1, DS_ROWS, DS_W), lambda bi, ii: (bi, 0, ii, 0)),
        ],
        out_shape=[
            jax.ShapeDtypeStruct((B, 1, UP_H, UP_W), f32),
            jax.ShapeDtypeStruct((B, 1, DS_H, DS_W), f32),
        ],
    )(xr, u_all, v_all, *wb_inputs,
      jnp.asarray(_S_NP), jnp.asarray(_U_NP), jnp.asarray(_WY_NP), WXr)
    return (out[0], out[1])
